# baseline (speedup 1.0000x reference)
import jax
import jax.numpy as jnp
from jax import lax
from jax.experimental import pallas as pl
from jax.experimental.pallas import tpu as pltpu
from jax.experimental.pallas import tpu_sc as plsc

BUFFER_SIZE = 10000
N_CLASSES = 100
BATCH = 4096
TAIL = BUFFER_SIZE - BATCH
ROW = 3 * 32 * 32

R1 = BATCH * ROW
TOT = BUFFER_SIZE * ROW
R2 = TOT - R1

NTILES = 32
S1 = R1 // NTILES
S2 = R2 // NTILES

CHUNK = 49_152


def _body(x_h, y_h, t_h, bx_h, by_h, bt_h, obx_h, oby_h, obt_h,
          spbuf, tfill, tailb, tvec, sem0, sem1):
    cid = lax.axis_index("c")
    sid = lax.axis_index("s")
    wid = sid * 2 + cid

    sems = (sem0, sem1)

    def copy_span(src_h, off0, sizes):
        n = len(sizes)
        offs = [off0]
        for s in sizes[:-1]:
            offs.append(offs[-1] + s)
        in_d = [None] * n
        out_d = [None] * n
        for i in range(n):
            b = i % 2
            if i >= 2:
                out_d[i - 2].wait()
            in_d[i] = pltpu.async_copy(
                src_h.at[pl.ds(offs[i], sizes[i])],
                spbuf.at[sid, b, pl.ds(0, sizes[i])], sems[b])
            in_d[i].wait()
            out_d[i] = pltpu.async_copy(
                spbuf.at[sid, b, pl.ds(0, sizes[i])],
                obx_h.at[pl.ds(offs[i], sizes[i])], sems[b])
        for i in range(max(0, n - 2), n):
            out_d[i].wait()

    copy_span(x_h, wid * S1, [CHUNK] * (S1 // CHUNK))
    n2, rem = divmod(S2, CHUNK)
    copy_span(bx_h, R1 + wid * S2, [CHUNK] * n2 + ([rem] if rem else []))

    @pl.when(wid == 30)
    def _():
        d0 = pltpu.async_copy(y_h, tfill, sem0)
        d1 = pltpu.async_copy(by_h.at[pl.ds(BATCH, TAIL)], tailb, sem1)
        d0.wait()
        pltpu.async_copy(tfill, oby_h.at[pl.ds(0, BATCH)], sem0).wait()
        d1.wait()
        pltpu.async_copy(tailb, oby_h.at[pl.ds(BATCH, TAIL)], sem1).wait()

    @pl.when(wid == 31)
    def _():
        pltpu.sync_copy(t_h, tvec)
        d1 = pltpu.async_copy(bt_h.at[pl.ds(BATCH, TAIL)], tailb, sem1)
        tv = tvec[...]
        for i in range(BATCH // 16):
            tfill[pl.ds(i * 16, 16)] = tv
        pltpu.async_copy(tfill, obt_h.at[pl.ds(0, BATCH)], sem0).wait()
        d1.wait()
        pltpu.async_copy(tailb, obt_h.at[pl.ds(BATCH, TAIL)], sem1).wait()


@jax.jit
def _er_update(x, y, t16, bx, by, bt):
    xf = x.reshape(R1)
    bxf = bx.reshape(TOT)
    mesh = plsc.VectorSubcoreMesh(core_axis_name="c", subcore_axis_name="s")
    obx, oby, obt = pl.kernel(
        _body,
        out_type=(
            jax.ShapeDtypeStruct((TOT,), jnp.float32),
            jax.ShapeDtypeStruct((BUFFER_SIZE,), jnp.int32),
            jax.ShapeDtypeStruct((BUFFER_SIZE,), jnp.int32),
        ),
        mesh=mesh,
        scratch_types=[
            pltpu.VMEM_SHARED((16, 2, CHUNK), jnp.float32),
            pltpu.VMEM((BATCH,), jnp.int32),
            pltpu.VMEM((TAIL,), jnp.int32),
            pltpu.VMEM((16,), jnp.int32),
            pltpu.SemaphoreType.DMA,
            pltpu.SemaphoreType.DMA,
        ],
    )(xf, y, t16, bxf, by, bt)
    return obx.reshape(bx.shape), oby, obt


def kernel(x, y, task_id, bx, by, bt):
    t16 = jnp.full((16,), task_id, dtype=jnp.int32)
    return _er_update(x, y, t16, bx, by, bt)
